# Initial kernel scaffold; baseline (speedup 1.0000x reference)
import jax
import jax.numpy as jnp
from jax.experimental import pallas as pl


def kernel(xyz):
    raise NotImplementedError("write your pallas kernel here")


# speedup vs baseline: 13.2263x; 13.2263x over previous
import functools

import jax
import jax.numpy as jnp
from jax.experimental import pallas as pl
from jax.experimental.pallas import tpu as pltpu

K = 16
QB = 256


def _knn_block(xq_ref, xr_ref, out_ref):
    xq = xq_ref[0]
    xr = xr_ref[0]
    sq_q = jnp.sum(xq * xq, axis=0)
    sq_r = jnp.sum(xr * xr, axis=0)
    inner = jax.lax.dot_general(
        xq, xr, (((0,), (0,)), ((), ())),
        preferred_element_type=jnp.float32)
    d2 = (sq_q[:, None] + sq_r[None, :]) - 2.0 * inner
    d2 = jnp.maximum(d2, 0.0)

    n = d2.shape[1]
    iota = jax.lax.broadcasted_iota(jnp.int32, d2.shape, 1)
    big = jnp.int32(n)
    inf = jnp.float32(jnp.inf)
    cols = []
    for _ in range(K):
        m = jnp.min(d2, axis=1, keepdims=True)
        masked_idx = jnp.where(d2 == m, iota, big)
        idx_k = jnp.min(masked_idx, axis=1)
        cols.append(idx_k)
        d2 = jnp.where(iota == idx_k[:, None], inf, d2)
    out_ref[0] = jnp.stack(cols, axis=1)


def kernel(xyz):
    b, n, _ = xyz.shape
    xyz_t = jnp.transpose(xyz, (0, 2, 1))
    grid = (b, n // QB)
    out = pl.pallas_call(
        _knn_block,
        grid=grid,
        in_specs=[
            pl.BlockSpec((1, 3, QB), lambda i, j: (i, 0, j)),
            pl.BlockSpec((1, 3, n), lambda i, j: (i, 0, 0)),
        ],
        out_specs=pl.BlockSpec((1, QB, K), lambda i, j: (i, j, 0)),
        out_shape=jax.ShapeDtypeStruct((b, n, K), jnp.int32),
    )(xyz_t, xyz_t)
    return out.astype(jnp.int64)
